# baseline (speedup 1.0000x reference)
import functools

import jax
import jax.numpy as jnp
from jax import lax
from jax.experimental import pallas as pl
from jax.experimental.pallas import tpu as pltpu
from jax.experimental.pallas import tpu_sc as plsc

N = 10000
N2 = 10240
E = 320000
D = 128
EPS = 1e-5
NC = 2
NS = 16
NW = NC * NS
E2 = 327680
EPW = E2 // NW
CH = 128
NCHUNK = EPW // CH
NBUF = 2
RPT = N2 // NS
RPT2 = N2 // NS


def _sc_mesh():
    return plsc.VectorSubcoreMesh(core_axis_name="c", subcore_axis_name="s")


def _deg_call(col, zeros_n1, ones_ch):
    dw = ones_ch.shape[1]
    @functools.partial(
        pl.kernel,
        out_type=jax.ShapeDtypeStruct((NC, N2, dw), jnp.float32),
        mesh=_sc_mesh(),
        scratch_types=[
            pltpu.VMEM((CH,), jnp.int32),
            pltpu.VMEM((CH, dw), jnp.float32),
            pltpu.VMEM_SHARED((N2, dw), jnp.float32),
        ],
    )
    def deg_kernel(col_hbm, zeros_hbm, ones_hbm, out_hbm, cidx_v, ones_v, deg_sh):
        c = lax.axis_index("c")
        s = lax.axis_index("s")
        wid = s * NC + c
        pltpu.sync_copy(ones_hbm, ones_v)
        pltpu.sync_copy(zeros_hbm.at[pl.ds(RPT2 * s, RPT2)],
                        deg_sh.at[pl.ds(RPT2 * s, RPT2)])
        plsc.subcore_barrier()

        def step(g, carry):
            eb = pl.multiple_of(wid * EPW + g * CH, 8)
            pltpu.sync_copy(col_hbm.at[pl.ds(eb, CH)], cidx_v)
            pltpu.sync_copy(ones_v, deg_sh.at[cidx_v], add=True)
            return carry

        lax.fori_loop(0, NCHUNK, step, 0)
        plsc.subcore_barrier()
        pltpu.sync_copy(deg_sh.at[pl.ds(RPT2 * s, RPT2)],
                        out_hbm.at[c, pl.ds(RPT2 * s, RPT2)])

    return deg_kernel(col, zeros_n1, ones_ch)


def _edge_call(hp, row, col, zeros_nd):
    @functools.partial(
        pl.kernel,
        out_type=jax.ShapeDtypeStruct((NC, N2, D), jnp.float32),
        mesh=_sc_mesh(),
        scratch_types=(
            [pltpu.VMEM((CH,), jnp.int32) for _ in range(NBUF)]
            + [pltpu.VMEM((CH,), jnp.int32) for _ in range(NBUF)]
            + [pltpu.VMEM((CH, D), jnp.float32) for _ in range(NBUF)]
            + [pltpu.VMEM_SHARED((N2, D), jnp.float32)]
            + [pltpu.SemaphoreType.DMA for _ in range(2 * NBUF)]
        ),
    )
    def edge_kernel(hp_hbm, row_hbm, col_hbm, zeros_hbm, out_hbm, *refs):
        ridx = refs[0:NBUF]
        cidx = refs[NBUF:2 * NBUF]
        rows = refs[2 * NBUF:3 * NBUF]
        acc_sh = refs[3 * NBUF]
        gsem = refs[3 * NBUF + 1:3 * NBUF + 1 + NBUF]
        ssem = refs[3 * NBUF + 1 + NBUF:3 * NBUF + 1 + 2 * NBUF]
        c = lax.axis_index("c")
        s = lax.axis_index("s")
        wid = s * NC + c
        base = wid * EPW
        pltpu.sync_copy(zeros_hbm.at[pl.ds(RPT * s, RPT)],
                        acc_sh.at[pl.ds(RPT * s, RPT)])
        plsc.subcore_barrier()

        def fetch(g, b):
            eb = pl.multiple_of(base + g * CH, 8)
            pltpu.sync_copy(row_hbm.at[pl.ds(eb, CH)], ridx[b])
            pltpu.sync_copy(col_hbm.at[pl.ds(eb, CH)], cidx[b])
            pltpu.async_copy(hp_hbm.at[ridx[b]], rows[b], gsem[b])

        def gwait(b):
            pltpu.make_async_copy(hp_hbm.at[ridx[b]], rows[b], gsem[b]).wait()

        def sstart(b):
            pltpu.async_copy(rows[b], acc_sh.at[cidx[b]], ssem[b], add=True)

        def swait(b):
            pltpu.make_async_copy(rows[b], acc_sh.at[cidx[b]], ssem[b]).wait()

        for b in range(NBUF - 1):
            fetch(b, b)

        def body(k, carry):
            g0 = NBUF * k
            for j in range(NBUF):
                gwait(j)
                sstart(j)
                tgt = (j + NBUF - 1) % NBUF
                if j == 0:
                    @pl.when(k >= 1)
                    def _():
                        swait(tgt)
                else:
                    swait(tgt)
                fetch(g0 + j + NBUF - 1, tgt)
            return carry

        lax.fori_loop(0, NCHUNK // NBUF - 1, body, 0)
        g0 = NCHUNK - NBUF
        for j in range(NBUF):
            b = (g0 + j) % NBUF
            gwait(b)
            sstart(b)
            if j == 0:
                tgt = (b + NBUF - 1) % NBUF
                swait(tgt)
                fetch(NCHUNK - 1, tgt)
        for j in range(NBUF):
            swait((g0 + j) % NBUF)
        plsc.subcore_barrier()
        pltpu.sync_copy(acc_sh.at[pl.ds(RPT * s, RPT)],
                        out_hbm.at[c, pl.ds(RPT * s, RPT)])

    return edge_kernel(hp, row, col, zeros_nd)


def _dis_from(degp_ref):
    deg = (degp_ref[0] + degp_ref[1])[:N, 0:1]
    return jnp.where(deg > 0.0, lax.rsqrt(deg), 0.0)


def _bn(y, g_ref, b_ref):
    mean = jnp.mean(y, axis=0, keepdims=True)
    var = jnp.mean((y - mean) ** 2, axis=0, keepdims=True)
    return (y - mean) * lax.rsqrt(var + EPS) * g_ref[...][None, :] + b_ref[...][None, :]


def _prep_body(h_ref, wg_ref, wl_ref, degp_ref, hp_ref, hl_ref):
    dis = _dis_from(degp_ref)
    h = h_ref[...]
    hp_ref[...] = jnp.dot(h, wg_ref[...], preferred_element_type=jnp.float32) * dis
    hl_ref[...] = jnp.dot(h, wl_ref[...], preferred_element_type=jnp.float32)


def _prep_call(h, wg, wl, degp):
    return pl.pallas_call(
        _prep_body,
        out_shape=(jax.ShapeDtypeStruct((N, D), jnp.float32),
                   jax.ShapeDtypeStruct((N, D), jnp.float32)),
    )(h, wg, wl, degp)


def _mid_body(hl_ref, acc_ref, degp_ref, g_ref, b_ref, wg_ref, wl_ref,
              hp_ref, hlo_ref):
    dis = _dis_from(degp_ref)
    y = hl_ref[...] + dis * (acc_ref[0] + acc_ref[1])[:N]
    h = jnp.maximum(_bn(y, g_ref, b_ref), 0.0)
    hp_ref[...] = jnp.dot(h, wg_ref[...], preferred_element_type=jnp.float32) * dis
    hlo_ref[...] = jnp.dot(h, wl_ref[...], preferred_element_type=jnp.float32)


def _mid_call(hl, acc, degp, gamma, beta, wg, wl):
    return pl.pallas_call(
        _mid_body,
        out_shape=(jax.ShapeDtypeStruct((N, D), jnp.float32),
                   jax.ShapeDtypeStruct((N, D), jnp.float32)),
    )(hl, acc, degp, gamma, beta, wg, wl)


def _fin_body(hl_ref, acc_ref, degp_ref, g_ref, b_ref, out_ref):
    dis = _dis_from(degp_ref)
    y = hl_ref[...] + dis * (acc_ref[0] + acc_ref[1])[:N]
    out_ref[...] = _bn(y, g_ref, b_ref)


def _fin_call(hl, acc, degp, gamma, beta):
    return pl.pallas_call(
        _fin_body,
        out_shape=jax.ShapeDtypeStruct((N, D), jnp.float32),
    )(hl, acc, degp, gamma, beta)


def kernel(x, edge_index, W_lin0, W_gcn0, gamma0, beta0,
           W_lin1, W_gcn1, gamma1, beta1):
    pad = E2 - E
    row = jnp.concatenate([edge_index[0], jnp.zeros((pad,), jnp.int32)])
    col = jnp.concatenate([edge_index[1], jnp.full((pad,), N2 - 1, jnp.int32)])
    zeros_nd = jnp.zeros((N2, D), jnp.float32)
    zeros_n1 = jnp.zeros((N2, 128), jnp.float32)
    ones_ch = jnp.ones((CH, 128), jnp.float32)

    degp = _deg_call(col, zeros_n1, ones_ch)
    hp0, hl0 = _prep_call(x, W_gcn0, W_lin0, degp)
    acc0 = _edge_call(hp0, row, col, zeros_nd)
    hp1, hl1 = _mid_call(hl0, acc0, degp, gamma0, beta0, W_gcn1, W_lin1)
    acc1 = _edge_call(hp1, row, col, zeros_nd)
    return _fin_call(hl1, acc1, degp, gamma1, beta1)


# speedup vs baseline: 1.0801x; 1.0801x over previous
import functools

import jax
import jax.numpy as jnp
from jax import lax
from jax.experimental import pallas as pl
from jax.experimental.pallas import tpu as pltpu
from jax.experimental.pallas import tpu_sc as plsc

N = 10000
N2 = 10240
E = 320000
D = 128
EPS = 1e-5
NC = 2
NS = 16
NW = NC * NS
E2 = 327680
EPW = E2 // NW
CH = 128
NCHUNK = EPW // CH
NBUF = 2
RPT = N2 // NS
RPT2 = N2 // NS


def _sc_mesh():
    return plsc.VectorSubcoreMesh(core_axis_name="c", subcore_axis_name="s")


def _deg_call(col, zeros_n1, ones_ch):
    dw = ones_ch.shape[1]
    @functools.partial(
        pl.kernel,
        out_type=jax.ShapeDtypeStruct((NC, N2, dw), jnp.float32),
        mesh=_sc_mesh(),
        scratch_types=[
            pltpu.VMEM((CH,), jnp.int32),
            pltpu.VMEM((CH, dw), jnp.float32),
            pltpu.VMEM_SHARED((N2, dw), jnp.float32),
        ],
    )
    def deg_kernel(col_hbm, zeros_hbm, ones_hbm, out_hbm, cidx_v, ones_v, deg_sh):
        c = lax.axis_index("c")
        s = lax.axis_index("s")
        wid = s * NC + c
        pltpu.sync_copy(ones_hbm, ones_v)
        pltpu.sync_copy(zeros_hbm.at[pl.ds(RPT2 * s, RPT2)],
                        deg_sh.at[pl.ds(RPT2 * s, RPT2)])
        plsc.subcore_barrier()

        def step(g, carry):
            eb = pl.multiple_of(wid * EPW + g * CH, 8)
            pltpu.sync_copy(col_hbm.at[pl.ds(eb, CH)], cidx_v)
            pltpu.sync_copy(ones_v, deg_sh.at[cidx_v], add=True)
            return carry

        lax.fori_loop(0, NCHUNK, step, 0)
        plsc.subcore_barrier()
        pltpu.sync_copy(deg_sh.at[pl.ds(RPT2 * s, RPT2)],
                        out_hbm.at[c, pl.ds(RPT2 * s, RPT2)])

    return deg_kernel(col, zeros_n1, ones_ch)


def _edge_call(hp, row, col, zeros_nd):
    @functools.partial(
        pl.kernel,
        out_type=jax.ShapeDtypeStruct((NC, N2, D), jnp.float32),
        mesh=_sc_mesh(),
        scratch_types=(
            [pltpu.VMEM((CH,), jnp.int32) for _ in range(NBUF)]
            + [pltpu.VMEM((CH,), jnp.int32) for _ in range(NBUF)]
            + [pltpu.VMEM((CH, D), jnp.float32) for _ in range(NBUF)]
            + [pltpu.VMEM_SHARED((N2, D), jnp.float32)]
            + [pltpu.SemaphoreType.DMA for _ in range(2 * NBUF)]
        ),
    )
    def edge_kernel(hp_hbm, row_hbm, col_hbm, zeros_hbm, out_hbm, *refs):
        ridx = refs[0:NBUF]
        cidx = refs[NBUF:2 * NBUF]
        rows = refs[2 * NBUF:3 * NBUF]
        acc_sh = refs[3 * NBUF]
        gsem = refs[3 * NBUF + 1:3 * NBUF + 1 + NBUF]
        ssem = refs[3 * NBUF + 1 + NBUF:3 * NBUF + 1 + 2 * NBUF]
        c = lax.axis_index("c")
        s = lax.axis_index("s")
        wid = s * NC + c
        base = wid * EPW
        pltpu.sync_copy(zeros_hbm.at[pl.ds(RPT * s, RPT)],
                        acc_sh.at[pl.ds(RPT * s, RPT)])
        plsc.subcore_barrier()

        def fetch(g, b):
            eb = pl.multiple_of(base + g * CH, 8)
            pltpu.sync_copy(row_hbm.at[pl.ds(eb, CH)], ridx[b])
            pltpu.sync_copy(col_hbm.at[pl.ds(eb, CH)], cidx[b])
            pltpu.async_copy(hp_hbm.at[ridx[b]], rows[b], gsem[b])

        def gwait(b):
            pltpu.make_async_copy(hp_hbm.at[ridx[b]], rows[b], gsem[b]).wait()

        def sstart(b):
            pltpu.async_copy(rows[b], acc_sh.at[cidx[b]], ssem[b], add=True)

        def swait(b):
            pltpu.make_async_copy(rows[b], acc_sh.at[cidx[b]], ssem[b]).wait()

        def drain(b):
            gwait(b)
            pltpu.sync_copy(rows[b], acc_sh.at[cidx[b]], add=True)

        fetch(0, 0)

        def body(k, carry):
            g0 = 2 * k
            fetch(g0 + 1, 1)
            drain(0)
            @pl.when(g0 + 2 < NCHUNK)
            def _():
                fetch(g0 + 2, 0)
            drain(1)
            return carry

        lax.fori_loop(0, NCHUNK // 2, body, 0)
        plsc.subcore_barrier()
        pltpu.sync_copy(acc_sh.at[pl.ds(RPT * s, RPT)],
                        out_hbm.at[c, pl.ds(RPT * s, RPT)])

    return edge_kernel(hp, row, col, zeros_nd)


def _dis_from(degp_ref):
    deg = (degp_ref[0] + degp_ref[1])[:N, 0:1]
    return jnp.where(deg > 0.0, lax.rsqrt(deg), 0.0)


def _bn(y, g_ref, b_ref):
    mean = jnp.mean(y, axis=0, keepdims=True)
    var = jnp.mean((y - mean) ** 2, axis=0, keepdims=True)
    return (y - mean) * lax.rsqrt(var + EPS) * g_ref[...][None, :] + b_ref[...][None, :]


def _prep_body(h_ref, wg_ref, wl_ref, degp_ref, hp_ref, hl_ref):
    dis = _dis_from(degp_ref)
    h = h_ref[...]
    hp_ref[...] = jnp.dot(h, wg_ref[...], preferred_element_type=jnp.float32) * dis
    hl_ref[...] = jnp.dot(h, wl_ref[...], preferred_element_type=jnp.float32)


def _prep_call(h, wg, wl, degp):
    return pl.pallas_call(
        _prep_body,
        out_shape=(jax.ShapeDtypeStruct((N, D), jnp.float32),
                   jax.ShapeDtypeStruct((N, D), jnp.float32)),
    )(h, wg, wl, degp)


def _mid_body(hl_ref, acc_ref, degp_ref, g_ref, b_ref, wg_ref, wl_ref,
              hp_ref, hlo_ref):
    dis = _dis_from(degp_ref)
    y = hl_ref[...] + dis * (acc_ref[0] + acc_ref[1])[:N]
    h = jnp.maximum(_bn(y, g_ref, b_ref), 0.0)
    hp_ref[...] = jnp.dot(h, wg_ref[...], preferred_element_type=jnp.float32) * dis
    hlo_ref[...] = jnp.dot(h, wl_ref[...], preferred_element_type=jnp.float32)


def _mid_call(hl, acc, degp, gamma, beta, wg, wl):
    return pl.pallas_call(
        _mid_body,
        out_shape=(jax.ShapeDtypeStruct((N, D), jnp.float32),
                   jax.ShapeDtypeStruct((N, D), jnp.float32)),
    )(hl, acc, degp, gamma, beta, wg, wl)


def _fin_body(hl_ref, acc_ref, degp_ref, g_ref, b_ref, out_ref):
    dis = _dis_from(degp_ref)
    y = hl_ref[...] + dis * (acc_ref[0] + acc_ref[1])[:N]
    out_ref[...] = _bn(y, g_ref, b_ref)


def _fin_call(hl, acc, degp, gamma, beta):
    return pl.pallas_call(
        _fin_body,
        out_shape=jax.ShapeDtypeStruct((N, D), jnp.float32),
    )(hl, acc, degp, gamma, beta)


def kernel(x, edge_index, W_lin0, W_gcn0, gamma0, beta0,
           W_lin1, W_gcn1, gamma1, beta1):
    pad = E2 - E
    row = jnp.concatenate([edge_index[0], jnp.zeros((pad,), jnp.int32)])
    col = jnp.concatenate([edge_index[1], jnp.full((pad,), N2 - 1, jnp.int32)])
    zeros_nd = jnp.zeros((N2, D), jnp.float32)
    zeros_n1 = jnp.zeros((N2, 128), jnp.float32)
    ones_ch = jnp.ones((CH, 128), jnp.float32)

    degp = _deg_call(col, zeros_n1, ones_ch)
    hp0, hl0 = _prep_call(x, W_gcn0, W_lin0, degp)
    acc0 = _edge_call(hp0, row, col, zeros_nd)
    hp1, hl1 = _mid_call(hl0, acc0, degp, gamma0, beta0, W_gcn1, W_lin1)
    acc1 = _edge_call(hp1, row, col, zeros_nd)
    return _fin_call(hl1, acc1, degp, gamma1, beta1)


# speedup vs baseline: 1.1044x; 1.0226x over previous
import functools

import jax
import jax.numpy as jnp
from jax import lax
from jax.experimental import pallas as pl
from jax.experimental.pallas import tpu as pltpu
from jax.experimental.pallas import tpu_sc as plsc

N = 10000
N2 = 10240
E = 320000
D = 128
EPS = 1e-5
NC = 2
NS = 16
NW = NC * NS
E2 = 327680
EPW = E2 // NW
CH = 80
NCHUNK = EPW // CH
NBUF = 2
RPT = N2 // NS
RPT2 = N2 // NS


def _sc_mesh():
    return plsc.VectorSubcoreMesh(core_axis_name="c", subcore_axis_name="s")


def _deg_call(col, zeros_n1, ones_ch):
    dw = ones_ch.shape[1]
    @functools.partial(
        pl.kernel,
        out_type=jax.ShapeDtypeStruct((NC, N2, dw), jnp.float32),
        mesh=_sc_mesh(),
        scratch_types=[
            pltpu.VMEM((CH,), jnp.int32),
            pltpu.VMEM((CH, dw), jnp.float32),
            pltpu.VMEM_SHARED((N2, dw), jnp.float32),
        ],
    )
    def deg_kernel(col_hbm, zeros_hbm, ones_hbm, out_hbm, cidx_v, ones_v, deg_sh):
        c = lax.axis_index("c")
        s = lax.axis_index("s")
        wid = s * NC + c
        pltpu.sync_copy(ones_hbm, ones_v)
        pltpu.sync_copy(zeros_hbm.at[pl.ds(RPT2 * s, RPT2)],
                        deg_sh.at[pl.ds(RPT2 * s, RPT2)])
        plsc.subcore_barrier()

        def step(g, carry):
            eb = pl.multiple_of(wid * EPW + g * CH, 8)
            pltpu.sync_copy(col_hbm.at[pl.ds(eb, CH)], cidx_v)
            pltpu.sync_copy(ones_v, deg_sh.at[cidx_v], add=True)
            return carry

        lax.fori_loop(0, NCHUNK, step, 0)
        plsc.subcore_barrier()
        pltpu.sync_copy(deg_sh.at[pl.ds(RPT2 * s, RPT2)],
                        out_hbm.at[c, pl.ds(RPT2 * s, RPT2)])

    return deg_kernel(col, zeros_n1, ones_ch)


def _edge_call(hp, row, col, zeros_nd):
    @functools.partial(
        pl.kernel,
        out_type=jax.ShapeDtypeStruct((NC, N2, D), jnp.float32),
        mesh=_sc_mesh(),
        scratch_types=(
            [pltpu.VMEM((CH,), jnp.int32) for _ in range(NBUF)]
            + [pltpu.VMEM((CH,), jnp.int32) for _ in range(NBUF)]
            + [pltpu.VMEM((CH, D), jnp.float32) for _ in range(NBUF)]
            + [pltpu.VMEM_SHARED((N2, D), jnp.float32)]
            + [pltpu.SemaphoreType.DMA for _ in range(2 * NBUF)]
        ),
    )
    def edge_kernel(hp_hbm, row_hbm, col_hbm, zeros_hbm, out_hbm, *refs):
        ridx = refs[0:NBUF]
        cidx = refs[NBUF:2 * NBUF]
        rows = refs[2 * NBUF:3 * NBUF]
        acc_sh = refs[3 * NBUF]
        gsem = refs[3 * NBUF + 1:3 * NBUF + 1 + NBUF]
        ssem = refs[3 * NBUF + 1 + NBUF:3 * NBUF + 1 + 2 * NBUF]
        c = lax.axis_index("c")
        s = lax.axis_index("s")
        wid = s * NC + c
        base = wid * EPW
        pltpu.sync_copy(zeros_hbm.at[pl.ds(RPT * s, RPT)],
                        acc_sh.at[pl.ds(RPT * s, RPT)])
        plsc.subcore_barrier()

        def fetch(g, b):
            eb = pl.multiple_of(base + g * CH, 8)
            pltpu.sync_copy(row_hbm.at[pl.ds(eb, CH)], ridx[b])
            pltpu.sync_copy(col_hbm.at[pl.ds(eb, CH)], cidx[b])
            pltpu.async_copy(hp_hbm.at[ridx[b]], rows[b], gsem[b])

        def gwait(b):
            pltpu.make_async_copy(hp_hbm.at[ridx[b]], rows[b], gsem[b]).wait()

        def sstart(b):
            pltpu.async_copy(rows[b], acc_sh.at[cidx[b]], ssem[b], add=True)

        def swait(b):
            pltpu.make_async_copy(rows[b], acc_sh.at[cidx[b]], ssem[b]).wait()

        def drain(b):
            gwait(b)
            pltpu.sync_copy(rows[b], acc_sh.at[cidx[b]], add=True)

        fetch(0, 0)

        def body(k, carry):
            g0 = 2 * k
            fetch(g0 + 1, 1)
            drain(0)
            @pl.when(g0 + 2 < NCHUNK)
            def _():
                fetch(g0 + 2, 0)
            drain(1)
            return carry

        lax.fori_loop(0, NCHUNK // 2, body, 0)
        plsc.subcore_barrier()
        pltpu.sync_copy(acc_sh.at[pl.ds(RPT * s, RPT)],
                        out_hbm.at[c, pl.ds(RPT * s, RPT)])

    return edge_kernel(hp, row, col, zeros_nd)


def _dis_from(degp_ref):
    deg = (degp_ref[0] + degp_ref[1])[:N, 0:1]
    return jnp.where(deg > 0.0, lax.rsqrt(deg), 0.0)


def _bn(y, g_ref, b_ref):
    mean = jnp.mean(y, axis=0, keepdims=True)
    var = jnp.mean((y - mean) ** 2, axis=0, keepdims=True)
    return (y - mean) * lax.rsqrt(var + EPS) * g_ref[...][None, :] + b_ref[...][None, :]


def _prep_body(h_ref, wg_ref, wl_ref, degp_ref, hp_ref, hl_ref):
    dis = _dis_from(degp_ref)
    h = h_ref[...]
    hp_ref[...] = jnp.dot(h, wg_ref[...], preferred_element_type=jnp.float32) * dis
    hl_ref[...] = jnp.dot(h, wl_ref[...], preferred_element_type=jnp.float32)


def _prep_call(h, wg, wl, degp):
    return pl.pallas_call(
        _prep_body,
        out_shape=(jax.ShapeDtypeStruct((N, D), jnp.float32),
                   jax.ShapeDtypeStruct((N, D), jnp.float32)),
    )(h, wg, wl, degp)


def _mid_body(hl_ref, acc_ref, degp_ref, g_ref, b_ref, wg_ref, wl_ref,
              hp_ref, hlo_ref):
    dis = _dis_from(degp_ref)
    y = hl_ref[...] + dis * (acc_ref[0] + acc_ref[1])[:N]
    h = jnp.maximum(_bn(y, g_ref, b_ref), 0.0)
    hp_ref[...] = jnp.dot(h, wg_ref[...], preferred_element_type=jnp.float32) * dis
    hlo_ref[...] = jnp.dot(h, wl_ref[...], preferred_element_type=jnp.float32)


def _mid_call(hl, acc, degp, gamma, beta, wg, wl):
    return pl.pallas_call(
        _mid_body,
        out_shape=(jax.ShapeDtypeStruct((N, D), jnp.float32),
                   jax.ShapeDtypeStruct((N, D), jnp.float32)),
    )(hl, acc, degp, gamma, beta, wg, wl)


def _fin_body(hl_ref, acc_ref, degp_ref, g_ref, b_ref, out_ref):
    dis = _dis_from(degp_ref)
    y = hl_ref[...] + dis * (acc_ref[0] + acc_ref[1])[:N]
    out_ref[...] = _bn(y, g_ref, b_ref)


def _fin_call(hl, acc, degp, gamma, beta):
    return pl.pallas_call(
        _fin_body,
        out_shape=jax.ShapeDtypeStruct((N, D), jnp.float32),
    )(hl, acc, degp, gamma, beta)


def kernel(x, edge_index, W_lin0, W_gcn0, gamma0, beta0,
           W_lin1, W_gcn1, gamma1, beta1):
    pad = E2 - E
    row = jnp.concatenate([edge_index[0], jnp.zeros((pad,), jnp.int32)])
    col = jnp.concatenate([edge_index[1], jnp.full((pad,), N2 - 1, jnp.int32)])
    zeros_nd = jnp.zeros((N2, D), jnp.float32)
    zeros_n1 = jnp.zeros((N2, 128), jnp.float32)
    ones_ch = jnp.ones((CH, 128), jnp.float32)

    degp = _deg_call(col, zeros_n1, ones_ch)
    hp0, hl0 = _prep_call(x, W_gcn0, W_lin0, degp)
    acc0 = _edge_call(hp0, row, col, zeros_nd)
    hp1, hl1 = _mid_call(hl0, acc0, degp, gamma0, beta0, W_gcn1, W_lin1)
    acc1 = _edge_call(hp1, row, col, zeros_nd)
    return _fin_call(hl1, acc1, degp, gamma1, beta1)


# speedup vs baseline: 1.1168x; 1.0112x over previous
import functools

import jax
import jax.numpy as jnp
from jax import lax
from jax.experimental import pallas as pl
from jax.experimental.pallas import tpu as pltpu
from jax.experimental.pallas import tpu_sc as plsc

N = 10000
N2 = 10240
E = 320000
D = 128
EPS = 1e-5
NC = 2
NS = 16
NW = NC * NS
E2 = 327680
EPW = E2 // NW
CH = 80
NCHUNK = EPW // CH
NBUF = 2
RPT = N2 // NS
RPT2 = N2 // NS


def _sc_mesh():
    return plsc.VectorSubcoreMesh(core_axis_name="c", subcore_axis_name="s")


def _deg_call(col, zeros_n1, ones_ch):
    dw = ones_ch.shape[1]
    @functools.partial(
        pl.kernel,
        out_type=jax.ShapeDtypeStruct((NC, N2, dw), jnp.float32),
        mesh=_sc_mesh(),
        scratch_types=[
            pltpu.VMEM((CH,), jnp.int32),
            pltpu.VMEM((CH, dw), jnp.float32),
            pltpu.VMEM_SHARED((N2, dw), jnp.float32),
        ],
    )
    def deg_kernel(col_hbm, zeros_hbm, ones_hbm, out_hbm, cidx_v, ones_v, deg_sh):
        c = lax.axis_index("c")
        s = lax.axis_index("s")
        wid = s * NC + c
        pltpu.sync_copy(ones_hbm, ones_v)
        pltpu.sync_copy(zeros_hbm.at[pl.ds(RPT2 * s, RPT2)],
                        deg_sh.at[pl.ds(RPT2 * s, RPT2)])
        plsc.subcore_barrier()

        def step(g, carry):
            eb = pl.multiple_of(wid * EPW + g * CH, 8)
            pltpu.sync_copy(col_hbm.at[pl.ds(eb, CH)], cidx_v)
            pltpu.sync_copy(ones_v, deg_sh.at[cidx_v], add=True)
            return carry

        lax.fori_loop(0, NCHUNK, step, 0)
        plsc.subcore_barrier()
        pltpu.sync_copy(deg_sh.at[pl.ds(RPT2 * s, RPT2)],
                        out_hbm.at[c, pl.ds(RPT2 * s, RPT2)])

    return deg_kernel(col, zeros_n1, ones_ch)


def _edge_call(hp, row, col, zeros_nd):
    @functools.partial(
        pl.kernel,
        out_type=jax.ShapeDtypeStruct((NC, N2, D), jnp.float32),
        mesh=_sc_mesh(),
        scratch_types=(
            [pltpu.VMEM((CH,), jnp.int32) for _ in range(NBUF)]
            + [pltpu.VMEM((CH,), jnp.int32) for _ in range(NBUF)]
            + [pltpu.VMEM((CH, D), jnp.float32) for _ in range(NBUF)]
            + [pltpu.VMEM_SHARED((N2, D), jnp.float32)]
            + [pltpu.SemaphoreType.DMA for _ in range(2 * NBUF)]
        ),
    )
    def edge_kernel(hp_hbm, row_hbm, col_hbm, zeros_hbm, out_hbm, *refs):
        ridx = refs[0:NBUF]
        cidx = refs[NBUF:2 * NBUF]
        rows = refs[2 * NBUF:3 * NBUF]
        acc_sh = refs[3 * NBUF]
        gsem = refs[3 * NBUF + 1:3 * NBUF + 1 + NBUF]
        ssem = refs[3 * NBUF + 1 + NBUF:3 * NBUF + 1 + 2 * NBUF]
        c = lax.axis_index("c")
        s = lax.axis_index("s")
        wid = s * NC + c
        base = wid * EPW
        pltpu.sync_copy(zeros_hbm.at[pl.ds(RPT * s, RPT)],
                        acc_sh.at[pl.ds(RPT * s, RPT)])
        plsc.subcore_barrier()

        def fetch(g, b):
            eb = pl.multiple_of(base + g * CH, 8)
            pltpu.sync_copy(row_hbm.at[pl.ds(eb, CH)], ridx[b])
            pltpu.sync_copy(col_hbm.at[pl.ds(eb, CH)], cidx[b])
            pltpu.async_copy(hp_hbm.at[ridx[b]], rows[b], gsem[b])

        def gwait(b):
            pltpu.make_async_copy(hp_hbm.at[ridx[b]], rows[b], gsem[b]).wait()

        def sstart(b):
            pltpu.async_copy(rows[b], acc_sh.at[cidx[b]], ssem[b], add=True)

        def swait(b):
            pltpu.make_async_copy(rows[b], acc_sh.at[cidx[b]], ssem[b]).wait()

        def drain(b):
            gwait(b)
            pltpu.sync_copy(rows[b], acc_sh.at[cidx[b]], add=True)

        fetch(0, 0)

        def body(k, carry):
            g0 = 2 * k
            fetch(g0 + 1, 1)
            drain(0)
            @pl.when(g0 + 2 < NCHUNK)
            def _():
                fetch(g0 + 2, 0)
            drain(1)
            return carry

        lax.fori_loop(0, NCHUNK // 2, body, 0)
        plsc.subcore_barrier()
        pltpu.sync_copy(acc_sh.at[pl.ds(RPT * s, RPT)],
                        out_hbm.at[c, pl.ds(RPT * s, RPT)])

    return edge_kernel(hp, row, col, zeros_nd)


def _dis_from(degp_ref):
    deg = (degp_ref[0] + degp_ref[1])[:N, 0:1]
    return jnp.where(deg > 0.0, lax.rsqrt(deg), 0.0)


def _bn(y, g_ref, b_ref):
    mean = jnp.mean(y, axis=0, keepdims=True)
    var = jnp.mean((y - mean) ** 2, axis=0, keepdims=True)
    return (y - mean) * lax.rsqrt(var + EPS) * g_ref[...][None, :] + b_ref[...][None, :]


def _prep_body(h_ref, wg_ref, wl_ref, degp_ref, hp_ref, hl_ref):
    dis = _dis_from(degp_ref)
    h = h_ref[...]
    hp_ref[...] = jnp.dot(h, wg_ref[...], preferred_element_type=jnp.float32) * dis
    hl_ref[...] = jnp.dot(h, wl_ref[...], preferred_element_type=jnp.float32)


def _prep_call(h, wg, wl, degp):
    return pl.pallas_call(
        _prep_body,
        out_shape=(jax.ShapeDtypeStruct((N, D), jnp.float32),
                   jax.ShapeDtypeStruct((N, D), jnp.float32)),
    )(h, wg, wl, degp)


def _mid_body(hl_ref, acc_ref, degp_ref, g_ref, b_ref, wg_ref, wl_ref,
              hp_ref, hlo_ref):
    dis = _dis_from(degp_ref)
    y = hl_ref[...] + dis * (acc_ref[0] + acc_ref[1])[:N]
    h = jnp.maximum(_bn(y, g_ref, b_ref), 0.0)
    hp_ref[...] = jnp.dot(h, wg_ref[...], preferred_element_type=jnp.float32) * dis
    hlo_ref[...] = jnp.dot(h, wl_ref[...], preferred_element_type=jnp.float32)


def _mid_call(hl, acc, degp, gamma, beta, wg, wl):
    return pl.pallas_call(
        _mid_body,
        out_shape=(jax.ShapeDtypeStruct((N, D), jnp.float32),
                   jax.ShapeDtypeStruct((N, D), jnp.float32)),
    )(hl, acc, degp, gamma, beta, wg, wl)


def _fin_body(hl_ref, acc_ref, degp_ref, g_ref, b_ref, out_ref):
    dis = _dis_from(degp_ref)
    y = hl_ref[...] + dis * (acc_ref[0] + acc_ref[1])[:N]
    out_ref[...] = _bn(y, g_ref, b_ref)


def _fin_call(hl, acc, degp, gamma, beta):
    return pl.pallas_call(
        _fin_body,
        out_shape=jax.ShapeDtypeStruct((N, D), jnp.float32),
    )(hl, acc, degp, gamma, beta)


def kernel(x, edge_index, W_lin0, W_gcn0, gamma0, beta0,
           W_lin1, W_gcn1, gamma1, beta1):
    pad = E2 - E
    row = jnp.concatenate([edge_index[0], jnp.zeros((pad,), jnp.int32)])
    dummy = N + (jnp.arange(pad, dtype=jnp.int32) % (N2 - N))
    col = jnp.concatenate([edge_index[1], dummy])
    zeros_nd = jnp.zeros((N2, D), jnp.float32)
    zeros_n1 = jnp.zeros((N2, 128), jnp.float32)
    ones_ch = jnp.ones((CH, 128), jnp.float32)

    degp = _deg_call(col, zeros_n1, ones_ch)
    hp0, hl0 = _prep_call(x, W_gcn0, W_lin0, degp)
    acc0 = _edge_call(hp0, row, col, zeros_nd)
    hp1, hl1 = _mid_call(hl0, acc0, degp, gamma0, beta0, W_gcn1, W_lin1)
    acc1 = _edge_call(hp1, row, col, zeros_nd)
    return _fin_call(hl1, acc1, degp, gamma1, beta1)


# speedup vs baseline: 2.2605x; 2.0241x over previous
import functools

import jax
import jax.numpy as jnp
from jax import lax
from jax.experimental import pallas as pl
from jax.experimental.pallas import tpu as pltpu
from jax.experimental.pallas import tpu_sc as plsc

N = 10000
N2 = 10240
E = 320000
D = 128
EPS = 1e-5
NC = 2
NS = 16
NW = NC * NS
E2 = 320000
EPW = E2 // NW
CH = 80
NCHUNK = EPW // CH
NBUF = 2
RPT = N2 // NS
RPT2 = N2 // NS


def _sc_mesh():
    return plsc.VectorSubcoreMesh(core_axis_name="c", subcore_axis_name="s")


def _deg_call(col, zeros_n1, ones_ch):
    dw = ones_ch.shape[1]
    @functools.partial(
        pl.kernel,
        out_type=jax.ShapeDtypeStruct((NC, N2, dw), jnp.float32),
        mesh=_sc_mesh(),
        scratch_types=[
            pltpu.VMEM((CH,), jnp.int32),
            pltpu.VMEM((CH, dw), jnp.float32),
            pltpu.VMEM_SHARED((N2, dw), jnp.float32),
        ],
    )
    def deg_kernel(col_hbm, zeros_hbm, ones_hbm, out_hbm, cidx_v, ones_v, deg_sh):
        c = lax.axis_index("c")
        s = lax.axis_index("s")
        wid = s * NC + c
        pltpu.sync_copy(ones_hbm, ones_v)
        pltpu.sync_copy(zeros_hbm.at[pl.ds(RPT2 * s, RPT2)],
                        deg_sh.at[pl.ds(RPT2 * s, RPT2)])
        plsc.subcore_barrier()

        def step(g, carry):
            eb = pl.multiple_of(wid * EPW + g * CH, 8)
            pltpu.sync_copy(col_hbm.at[pl.ds(eb, CH)], cidx_v)
            pltpu.sync_copy(ones_v, deg_sh.at[cidx_v], add=True)
            return carry

        lax.fori_loop(0, NCHUNK, step, 0)
        plsc.subcore_barrier()
        pltpu.sync_copy(deg_sh.at[pl.ds(RPT2 * s, RPT2)],
                        out_hbm.at[c, pl.ds(RPT2 * s, RPT2)])

    return deg_kernel(col, zeros_n1, ones_ch)


def _edge_call(hp, row, col, zeros_nd):
    @functools.partial(
        pl.kernel,
        out_type=jax.ShapeDtypeStruct((NC, N2, D), jnp.float32),
        mesh=_sc_mesh(),
        scratch_types=(
            [pltpu.VMEM((CH,), jnp.int32) for _ in range(NBUF)]
            + [pltpu.VMEM((CH,), jnp.int32) for _ in range(NBUF)]
            + [pltpu.VMEM((CH, D), jnp.float32) for _ in range(NBUF)]
            + [pltpu.VMEM_SHARED((N2, D), jnp.float32)]
            + [pltpu.SemaphoreType.DMA for _ in range(2 * NBUF)]
        ),
    )
    def edge_kernel(hp_hbm, row_hbm, col_hbm, zeros_hbm, out_hbm, *refs):
        ridx = refs[0:NBUF]
        cidx = refs[NBUF:2 * NBUF]
        rows = refs[2 * NBUF:3 * NBUF]
        acc_sh = refs[3 * NBUF]
        gsem = refs[3 * NBUF + 1:3 * NBUF + 1 + NBUF]
        ssem = refs[3 * NBUF + 1 + NBUF:3 * NBUF + 1 + 2 * NBUF]
        c = lax.axis_index("c")
        s = lax.axis_index("s")
        wid = s * NC + c
        base = wid * EPW
        pltpu.sync_copy(zeros_hbm.at[pl.ds(RPT * s, RPT)],
                        acc_sh.at[pl.ds(RPT * s, RPT)])
        plsc.subcore_barrier()

        def fetch(g, b):
            eb = pl.multiple_of(base + g * CH, 8)
            pltpu.sync_copy(row_hbm.at[pl.ds(eb, CH)], ridx[b])
            pltpu.sync_copy(col_hbm.at[pl.ds(eb, CH)], cidx[b])
            pltpu.async_copy(hp_hbm.at[ridx[b]], rows[b], gsem[b])

        def gwait(b):
            pltpu.make_async_copy(hp_hbm.at[ridx[b]], rows[b], gsem[b]).wait()

        def sstart(b):
            pltpu.async_copy(rows[b], acc_sh.at[cidx[b]], ssem[b], add=True)

        def swait(b):
            pltpu.make_async_copy(rows[b], acc_sh.at[cidx[b]], ssem[b]).wait()

        def drain(b):
            gwait(b)
            pltpu.sync_copy(rows[b], acc_sh.at[cidx[b]], add=True)

        fetch(0, 0)

        def body(k, carry):
            g0 = 2 * k
            fetch(g0 + 1, 1)
            drain(0)
            @pl.when(g0 + 2 < NCHUNK)
            def _():
                fetch(g0 + 2, 0)
            drain(1)
            return carry

        lax.fori_loop(0, NCHUNK // 2, body, 0)
        if NCHUNK % 2 == 1:
            drain(0)
        plsc.subcore_barrier()
        pltpu.sync_copy(acc_sh.at[pl.ds(RPT * s, RPT)],
                        out_hbm.at[c, pl.ds(RPT * s, RPT)])

    return edge_kernel(hp, row, col, zeros_nd)


def _dis_from(degp_ref):
    deg = (degp_ref[0] + degp_ref[1])[:N, 0:1]
    return jnp.where(deg > 0.0, lax.rsqrt(deg), 0.0)


def _bn(y, g_ref, b_ref):
    mean = jnp.mean(y, axis=0, keepdims=True)
    var = jnp.mean((y - mean) ** 2, axis=0, keepdims=True)
    return (y - mean) * lax.rsqrt(var + EPS) * g_ref[...][None, :] + b_ref[...][None, :]


def _prep_body(h_ref, wg_ref, wl_ref, degp_ref, hp_ref, hl_ref):
    dis = _dis_from(degp_ref)
    h = h_ref[...]
    hp_ref[...] = jnp.dot(h, wg_ref[...], preferred_element_type=jnp.float32) * dis
    hl_ref[...] = jnp.dot(h, wl_ref[...], preferred_element_type=jnp.float32)


def _prep_call(h, wg, wl, degp):
    return pl.pallas_call(
        _prep_body,
        out_shape=(jax.ShapeDtypeStruct((N, D), jnp.float32),
                   jax.ShapeDtypeStruct((N, D), jnp.float32)),
    )(h, wg, wl, degp)


def _mid_body(hl_ref, acc_ref, degp_ref, g_ref, b_ref, wg_ref, wl_ref,
              hp_ref, hlo_ref):
    dis = _dis_from(degp_ref)
    y = hl_ref[...] + dis * (acc_ref[0] + acc_ref[1])[:N]
    h = jnp.maximum(_bn(y, g_ref, b_ref), 0.0)
    hp_ref[...] = jnp.dot(h, wg_ref[...], preferred_element_type=jnp.float32) * dis
    hlo_ref[...] = jnp.dot(h, wl_ref[...], preferred_element_type=jnp.float32)


def _mid_call(hl, acc, degp, gamma, beta, wg, wl):
    return pl.pallas_call(
        _mid_body,
        out_shape=(jax.ShapeDtypeStruct((N, D), jnp.float32),
                   jax.ShapeDtypeStruct((N, D), jnp.float32)),
    )(hl, acc, degp, gamma, beta, wg, wl)


def _fin_body(hl_ref, acc_ref, degp_ref, g_ref, b_ref, out_ref):
    dis = _dis_from(degp_ref)
    y = hl_ref[...] + dis * (acc_ref[0] + acc_ref[1])[:N]
    out_ref[...] = _bn(y, g_ref, b_ref)


def _fin_call(hl, acc, degp, gamma, beta):
    return pl.pallas_call(
        _fin_body,
        out_shape=jax.ShapeDtypeStruct((N, D), jnp.float32),
    )(hl, acc, degp, gamma, beta)


def kernel(x, edge_index, W_lin0, W_gcn0, gamma0, beta0,
           W_lin1, W_gcn1, gamma1, beta1):
    pad = E2 - E
    row = jnp.concatenate([edge_index[0], jnp.zeros((pad,), jnp.int32)])
    dummy = N + (jnp.arange(pad, dtype=jnp.int32) % (N2 - N))
    col = jnp.concatenate([edge_index[1], dummy])
    zeros_nd = jnp.zeros((N2, D), jnp.float32)
    zeros_n1 = jnp.zeros((N2, 128), jnp.float32)
    ones_ch = jnp.ones((CH, 128), jnp.float32)

    degp = _deg_call(col, zeros_n1, ones_ch)
    hp0, hl0 = _prep_call(x, W_gcn0, W_lin0, degp)
    acc0 = _edge_call(hp0, row, col, zeros_nd)
    hp1, hl1 = _mid_call(hl0, acc0, degp, gamma0, beta0, W_gcn1, W_lin1)
    acc1 = _edge_call(hp1, row, col, zeros_nd)
    return _fin_call(hl1, acc1, degp, gamma1, beta1)


# speedup vs baseline: 2.7236x; 1.2049x over previous
import functools

import jax
import jax.numpy as jnp
from jax import lax
from jax.experimental import pallas as pl
from jax.experimental.pallas import tpu as pltpu
from jax.experimental.pallas import tpu_sc as plsc

N = 10000
N2 = 10240
E = 320000
D = 128
EPS = 1e-5
NC = 2
NS = 16
NW = NC * NS
E2 = 320000
EPW = E2 // NW
CH = 80
NCHUNK = EPW // CH
NBUF = 4
RPT = N2 // NS
RPT2 = N2 // NS


def _sc_mesh():
    return plsc.VectorSubcoreMesh(core_axis_name="c", subcore_axis_name="s")


def _deg_call(col, zeros_n1, ones_ch):
    dw = ones_ch.shape[1]
    @functools.partial(
        pl.kernel,
        out_type=jax.ShapeDtypeStruct((NC, N2, dw), jnp.float32),
        mesh=_sc_mesh(),
        scratch_types=[
            pltpu.VMEM((CH,), jnp.int32),
            pltpu.VMEM((CH, dw), jnp.float32),
            pltpu.VMEM_SHARED((N2, dw), jnp.float32),
        ],
    )
    def deg_kernel(col_hbm, zeros_hbm, ones_hbm, out_hbm, cidx_v, ones_v, deg_sh):
        c = lax.axis_index("c")
        s = lax.axis_index("s")
        wid = s * NC + c
        pltpu.sync_copy(ones_hbm, ones_v)
        pltpu.sync_copy(zeros_hbm.at[pl.ds(RPT2 * s, RPT2)],
                        deg_sh.at[pl.ds(RPT2 * s, RPT2)])
        plsc.subcore_barrier()

        def step(g, carry):
            eb = pl.multiple_of(wid * EPW + g * CH, 8)
            pltpu.sync_copy(col_hbm.at[pl.ds(eb, CH)], cidx_v)
            pltpu.sync_copy(ones_v, deg_sh.at[cidx_v], add=True)
            return carry

        lax.fori_loop(0, NCHUNK, step, 0)
        plsc.subcore_barrier()
        pltpu.sync_copy(deg_sh.at[pl.ds(RPT2 * s, RPT2)],
                        out_hbm.at[c, pl.ds(RPT2 * s, RPT2)])

    return deg_kernel(col, zeros_n1, ones_ch)


def _edge_call(hp, row, col, zeros_nd):
    @functools.partial(
        pl.kernel,
        out_type=jax.ShapeDtypeStruct((NC, N2, D), jnp.float32),
        mesh=_sc_mesh(),
        scratch_types=(
            [pltpu.VMEM((CH,), jnp.int32) for _ in range(NBUF)]
            + [pltpu.VMEM((CH,), jnp.int32) for _ in range(NBUF)]
            + [pltpu.VMEM((CH, D), jnp.float32) for _ in range(NBUF)]
            + [pltpu.VMEM_SHARED((N2, D), jnp.float32)]
            + [pltpu.SemaphoreType.DMA for _ in range(2 * NBUF)]
        ),
    )
    def edge_kernel(hp_hbm, row_hbm, col_hbm, zeros_hbm, out_hbm, *refs):
        ridx = refs[0:NBUF]
        cidx = refs[NBUF:2 * NBUF]
        rows = refs[2 * NBUF:3 * NBUF]
        acc_sh = refs[3 * NBUF]
        gsem = refs[3 * NBUF + 1:3 * NBUF + 1 + NBUF]
        ssem = refs[3 * NBUF + 1 + NBUF:3 * NBUF + 1 + 2 * NBUF]
        c = lax.axis_index("c")
        s = lax.axis_index("s")
        wid = s * NC + c
        base = wid * EPW
        pltpu.sync_copy(zeros_hbm.at[pl.ds(RPT * s, RPT)],
                        acc_sh.at[pl.ds(RPT * s, RPT)])
        plsc.subcore_barrier()

        def fetch(g, b):
            eb = pl.multiple_of(base + g * CH, 8)
            pltpu.sync_copy(row_hbm.at[pl.ds(eb, CH)], ridx[b])
            pltpu.sync_copy(col_hbm.at[pl.ds(eb, CH)], cidx[b])
            pltpu.async_copy(hp_hbm.at[ridx[b]], rows[b], gsem[b])

        def gwait(b):
            pltpu.make_async_copy(hp_hbm.at[ridx[b]], rows[b], gsem[b]).wait()

        def sstart(b):
            pltpu.async_copy(rows[b], acc_sh.at[cidx[b]], ssem[b], add=True)

        def swait(b):
            pltpu.make_async_copy(rows[b], acc_sh.at[cidx[b]], ssem[b]).wait()

        for b in range(NBUF - 1):
            fetch(b, b)

        def body(k, carry):
            g0 = NBUF * k
            for j in range(NBUF):
                g = g0 + j
                gwait(j)
                sstart(j)
                tgt = (j + NBUF - 1) % NBUF
                if j == 0:
                    @pl.when(k == 0)
                    def _():
                        fetch(NBUF - 1, tgt)

                    @pl.when(k >= 1)
                    def _():
                        swait(tgt)
                        fetch(g + NBUF - 1, tgt)
                else:
                    @pl.when(g + NBUF - 1 < NCHUNK)
                    def _():
                        swait(tgt)
                        fetch(g + NBUF - 1, tgt)
            return carry

        lax.fori_loop(0, NCHUNK // NBUF, body, 0)
        for t in range(NCHUNK % NBUF):
            g = (NCHUNK // NBUF) * NBUF + t
            gwait(g % NBUF)
            sstart(g % NBUF)
        for g in range(NCHUNK - NBUF, NCHUNK):
            swait(g % NBUF)
        plsc.subcore_barrier()
        pltpu.sync_copy(acc_sh.at[pl.ds(RPT * s, RPT)],
                        out_hbm.at[c, pl.ds(RPT * s, RPT)])

    return edge_kernel(hp, row, col, zeros_nd)


def _dis_from(degp_ref):
    deg = (degp_ref[0] + degp_ref[1])[:N, 0:1]
    return jnp.where(deg > 0.0, lax.rsqrt(deg), 0.0)


def _bn(y, g_ref, b_ref):
    mean = jnp.mean(y, axis=0, keepdims=True)
    var = jnp.mean((y - mean) ** 2, axis=0, keepdims=True)
    return (y - mean) * lax.rsqrt(var + EPS) * g_ref[...][None, :] + b_ref[...][None, :]


def _prep_body(h_ref, wg_ref, wl_ref, degp_ref, hp_ref, hl_ref):
    dis = _dis_from(degp_ref)
    h = h_ref[...]
    hp_ref[...] = jnp.dot(h, wg_ref[...], preferred_element_type=jnp.float32) * dis
    hl_ref[...] = jnp.dot(h, wl_ref[...], preferred_element_type=jnp.float32)


def _prep_call(h, wg, wl, degp):
    return pl.pallas_call(
        _prep_body,
        out_shape=(jax.ShapeDtypeStruct((N, D), jnp.float32),
                   jax.ShapeDtypeStruct((N, D), jnp.float32)),
    )(h, wg, wl, degp)


def _mid_body(hl_ref, acc_ref, degp_ref, g_ref, b_ref, wg_ref, wl_ref,
              hp_ref, hlo_ref):
    dis = _dis_from(degp_ref)
    y = hl_ref[...] + dis * (acc_ref[0] + acc_ref[1])[:N]
    h = jnp.maximum(_bn(y, g_ref, b_ref), 0.0)
    hp_ref[...] = jnp.dot(h, wg_ref[...], preferred_element_type=jnp.float32) * dis
    hlo_ref[...] = jnp.dot(h, wl_ref[...], preferred_element_type=jnp.float32)


def _mid_call(hl, acc, degp, gamma, beta, wg, wl):
    return pl.pallas_call(
        _mid_body,
        out_shape=(jax.ShapeDtypeStruct((N, D), jnp.float32),
                   jax.ShapeDtypeStruct((N, D), jnp.float32)),
    )(hl, acc, degp, gamma, beta, wg, wl)


def _fin_body(hl_ref, acc_ref, degp_ref, g_ref, b_ref, out_ref):
    dis = _dis_from(degp_ref)
    y = hl_ref[...] + dis * (acc_ref[0] + acc_ref[1])[:N]
    out_ref[...] = _bn(y, g_ref, b_ref)


def _fin_call(hl, acc, degp, gamma, beta):
    return pl.pallas_call(
        _fin_body,
        out_shape=jax.ShapeDtypeStruct((N, D), jnp.float32),
    )(hl, acc, degp, gamma, beta)


def kernel(x, edge_index, W_lin0, W_gcn0, gamma0, beta0,
           W_lin1, W_gcn1, gamma1, beta1):
    pad = E2 - E
    row = jnp.concatenate([edge_index[0], jnp.zeros((pad,), jnp.int32)])
    dummy = N + (jnp.arange(pad, dtype=jnp.int32) % (N2 - N))
    col = jnp.concatenate([edge_index[1], dummy])
    zeros_nd = jnp.zeros((N2, D), jnp.float32)
    zeros_n1 = jnp.zeros((N2, 128), jnp.float32)
    ones_ch = jnp.ones((CH, 128), jnp.float32)

    degp = _deg_call(col, zeros_n1, ones_ch)
    hp0, hl0 = _prep_call(x, W_gcn0, W_lin0, degp)
    acc0 = _edge_call(hp0, row, col, zeros_nd)
    hp1, hl1 = _mid_call(hl0, acc0, degp, gamma0, beta0, W_gcn1, W_lin1)
    acc1 = _edge_call(hp1, row, col, zeros_nd)
    return _fin_call(hl1, acc1, degp, gamma1, beta1)


# speedup vs baseline: 3.0445x; 1.1178x over previous
import functools

import jax
import jax.numpy as jnp
from jax import lax
from jax.experimental import pallas as pl
from jax.experimental.pallas import tpu as pltpu
from jax.experimental.pallas import tpu_sc as plsc

N = 10000
N2 = 10240
E = 320000
D = 128
EPS = 1e-5
NC = 2
NS = 16
NW = NC * NS
E2 = 320000
EPW = E2 // NW
CH = 80
NCHUNK = EPW // CH
NBUF = 4
RPT = N2 // NS
RPT2 = N2 // NS


def _sc_mesh():
    return plsc.VectorSubcoreMesh(core_axis_name="c", subcore_axis_name="s")


def _deg_call(col, zeros_n1, ones_ch):
    dw = ones_ch.shape[1]
    @functools.partial(
        pl.kernel,
        out_type=jax.ShapeDtypeStruct((NC, N2, dw), jnp.float32),
        mesh=_sc_mesh(),
        scratch_types=(
            [pltpu.VMEM((CH,), jnp.int32) for _ in range(NBUF)]
            + [pltpu.VMEM((CH, dw), jnp.float32),
               pltpu.VMEM_SHARED((N2, dw), jnp.float32)]
            + [pltpu.SemaphoreType.DMA for _ in range(NBUF)]
        ),
    )
    def deg_kernel(col_hbm, zeros_hbm, ones_hbm, out_hbm, *refs):
        cidx = refs[0:NBUF]
        ones_v = refs[NBUF]
        deg_sh = refs[NBUF + 1]
        ssem = refs[NBUF + 2:NBUF + 2 + NBUF]
        c = lax.axis_index("c")
        s = lax.axis_index("s")
        wid = s * NC + c
        base = wid * EPW
        pltpu.sync_copy(ones_hbm, ones_v)
        pltpu.sync_copy(zeros_hbm.at[pl.ds(RPT2 * s, RPT2)],
                        deg_sh.at[pl.ds(RPT2 * s, RPT2)])
        plsc.subcore_barrier()

        def fetch(g, b):
            eb = pl.multiple_of(base + g * CH, 8)
            pltpu.sync_copy(col_hbm.at[pl.ds(eb, CH)], cidx[b])

        def sstart(b):
            pltpu.async_copy(ones_v, deg_sh.at[cidx[b]], ssem[b], add=True)

        def swait(b):
            pltpu.make_async_copy(ones_v, deg_sh.at[cidx[b]], ssem[b]).wait()

        for b in range(NBUF - 1):
            fetch(b, b)

        def body(k, carry):
            g0 = NBUF * k
            for j in range(NBUF):
                g = g0 + j
                sstart(j)
                tgt = (j + NBUF - 1) % NBUF
                if j == 0:
                    @pl.when(k == 0)
                    def _():
                        fetch(NBUF - 1, tgt)

                    @pl.when(k >= 1)
                    def _():
                        swait(tgt)
                        fetch(g + NBUF - 1, tgt)
                else:
                    @pl.when(g + NBUF - 1 < NCHUNK)
                    def _():
                        swait(tgt)
                        fetch(g + NBUF - 1, tgt)
            return carry

        lax.fori_loop(0, NCHUNK // NBUF, body, 0)
        for t in range(NCHUNK % NBUF):
            g = (NCHUNK // NBUF) * NBUF + t
            sstart(g % NBUF)
        for g in range(NCHUNK - NBUF, NCHUNK):
            swait(g % NBUF)
        plsc.subcore_barrier()
        pltpu.sync_copy(deg_sh.at[pl.ds(RPT2 * s, RPT2)],
                        out_hbm.at[c, pl.ds(RPT2 * s, RPT2)])

    return deg_kernel(col, zeros_n1, ones_ch)


def _edge_call(hp, row, col, zeros_nd):
    @functools.partial(
        pl.kernel,
        out_type=jax.ShapeDtypeStruct((NC, N2, D), jnp.float32),
        mesh=_sc_mesh(),
        scratch_types=(
            [pltpu.VMEM((CH,), jnp.int32) for _ in range(NBUF)]
            + [pltpu.VMEM((CH,), jnp.int32) for _ in range(NBUF)]
            + [pltpu.VMEM((CH, D), jnp.float32) for _ in range(NBUF)]
            + [pltpu.VMEM_SHARED((N2, D), jnp.float32)]
            + [pltpu.SemaphoreType.DMA for _ in range(2 * NBUF)]
        ),
    )
    def edge_kernel(hp_hbm, row_hbm, col_hbm, zeros_hbm, out_hbm, *refs):
        ridx = refs[0:NBUF]
        cidx = refs[NBUF:2 * NBUF]
        rows = refs[2 * NBUF:3 * NBUF]
        acc_sh = refs[3 * NBUF]
        gsem = refs[3 * NBUF + 1:3 * NBUF + 1 + NBUF]
        ssem = refs[3 * NBUF + 1 + NBUF:3 * NBUF + 1 + 2 * NBUF]
        c = lax.axis_index("c")
        s = lax.axis_index("s")
        wid = s * NC + c
        base = wid * EPW
        pltpu.sync_copy(zeros_hbm.at[pl.ds(RPT * s, RPT)],
                        acc_sh.at[pl.ds(RPT * s, RPT)])
        plsc.subcore_barrier()

        def fetch(g, b):
            eb = pl.multiple_of(base + g * CH, 8)
            pltpu.sync_copy(row_hbm.at[pl.ds(eb, CH)], ridx[b])
            pltpu.sync_copy(col_hbm.at[pl.ds(eb, CH)], cidx[b])
            pltpu.async_copy(hp_hbm.at[ridx[b]], rows[b], gsem[b])

        def gwait(b):
            pltpu.make_async_copy(hp_hbm.at[ridx[b]], rows[b], gsem[b]).wait()

        def sstart(b):
            pltpu.async_copy(rows[b], acc_sh.at[cidx[b]], ssem[b], add=True)

        def swait(b):
            pltpu.make_async_copy(rows[b], acc_sh.at[cidx[b]], ssem[b]).wait()

        for b in range(NBUF - 1):
            fetch(b, b)

        def body(k, carry):
            g0 = NBUF * k
            for j in range(NBUF):
                g = g0 + j
                gwait(j)
                sstart(j)
                tgt = (j + NBUF - 1) % NBUF
                if j == 0:
                    @pl.when(k == 0)
                    def _():
                        fetch(NBUF - 1, tgt)

                    @pl.when(k >= 1)
                    def _():
                        swait(tgt)
                        fetch(g + NBUF - 1, tgt)
                else:
                    @pl.when(g + NBUF - 1 < NCHUNK)
                    def _():
                        swait(tgt)
                        fetch(g + NBUF - 1, tgt)
            return carry

        lax.fori_loop(0, NCHUNK // NBUF, body, 0)
        for t in range(NCHUNK % NBUF):
            g = (NCHUNK // NBUF) * NBUF + t
            gwait(g % NBUF)
            sstart(g % NBUF)
        for g in range(NCHUNK - NBUF, NCHUNK):
            swait(g % NBUF)
        plsc.subcore_barrier()
        pltpu.sync_copy(acc_sh.at[pl.ds(RPT * s, RPT)],
                        out_hbm.at[c, pl.ds(RPT * s, RPT)])

    return edge_kernel(hp, row, col, zeros_nd)


def _dis_from(degp_ref):
    deg = (degp_ref[0] + degp_ref[1])[:N, 0:1]
    return jnp.where(deg > 0.0, lax.rsqrt(deg), 0.0)


def _bn(y, g_ref, b_ref):
    mean = jnp.mean(y, axis=0, keepdims=True)
    var = jnp.mean((y - mean) ** 2, axis=0, keepdims=True)
    return (y - mean) * lax.rsqrt(var + EPS) * g_ref[...][None, :] + b_ref[...][None, :]


def _prep_body(h_ref, wg_ref, wl_ref, degp_ref, hp_ref, hl_ref):
    dis = _dis_from(degp_ref)
    h = h_ref[...]
    hp_ref[...] = jnp.dot(h, wg_ref[...], preferred_element_type=jnp.float32) * dis
    hl_ref[...] = jnp.dot(h, wl_ref[...], preferred_element_type=jnp.float32)


def _prep_call(h, wg, wl, degp):
    return pl.pallas_call(
        _prep_body,
        out_shape=(jax.ShapeDtypeStruct((N, D), jnp.float32),
                   jax.ShapeDtypeStruct((N, D), jnp.float32)),
    )(h, wg, wl, degp)


def _mid_body(hl_ref, acc_ref, degp_ref, g_ref, b_ref, wg_ref, wl_ref,
              hp_ref, hlo_ref):
    dis = _dis_from(degp_ref)
    y = hl_ref[...] + dis * (acc_ref[0] + acc_ref[1])[:N]
    h = jnp.maximum(_bn(y, g_ref, b_ref), 0.0)
    hp_ref[...] = jnp.dot(h, wg_ref[...], preferred_element_type=jnp.float32) * dis
    hlo_ref[...] = jnp.dot(h, wl_ref[...], preferred_element_type=jnp.float32)


def _mid_call(hl, acc, degp, gamma, beta, wg, wl):
    return pl.pallas_call(
        _mid_body,
        out_shape=(jax.ShapeDtypeStruct((N, D), jnp.float32),
                   jax.ShapeDtypeStruct((N, D), jnp.float32)),
    )(hl, acc, degp, gamma, beta, wg, wl)


def _fin_body(hl_ref, acc_ref, degp_ref, g_ref, b_ref, out_ref):
    dis = _dis_from(degp_ref)
    y = hl_ref[...] + dis * (acc_ref[0] + acc_ref[1])[:N]
    out_ref[...] = _bn(y, g_ref, b_ref)


def _fin_call(hl, acc, degp, gamma, beta):
    return pl.pallas_call(
        _fin_body,
        out_shape=jax.ShapeDtypeStruct((N, D), jnp.float32),
    )(hl, acc, degp, gamma, beta)


def kernel(x, edge_index, W_lin0, W_gcn0, gamma0, beta0,
           W_lin1, W_gcn1, gamma1, beta1):
    pad = E2 - E
    row = jnp.concatenate([edge_index[0], jnp.zeros((pad,), jnp.int32)])
    dummy = N + (jnp.arange(pad, dtype=jnp.int32) % (N2 - N))
    col = jnp.concatenate([edge_index[1], dummy])
    zeros_nd = jnp.zeros((N2, D), jnp.float32)
    zeros_n1 = jnp.zeros((N2, 128), jnp.float32)
    ones_ch = jnp.ones((CH, 128), jnp.float32)

    degp = _deg_call(col, zeros_n1, ones_ch)
    hp0, hl0 = _prep_call(x, W_gcn0, W_lin0, degp)
    acc0 = _edge_call(hp0, row, col, zeros_nd)
    hp1, hl1 = _mid_call(hl0, acc0, degp, gamma0, beta0, W_gcn1, W_lin1)
    acc1 = _edge_call(hp1, row, col, zeros_nd)
    return _fin_call(hl1, acc1, degp, gamma1, beta1)


# speedup vs baseline: 3.0567x; 1.0040x over previous
import functools

import jax
import jax.numpy as jnp
from jax import lax
from jax.experimental import pallas as pl
from jax.experimental.pallas import tpu as pltpu
from jax.experimental.pallas import tpu_sc as plsc

N = 10000
N2 = 10240
E = 320000
D = 128
EPS = 1e-5
NC = 2
NS = 16
NW = NC * NS
E2 = 320000
EPW = E2 // NW
CH = 80
NCHUNK = EPW // CH
NBUF = 4
RPT = N2 // NS
RPT2 = N2 // NS


def _sc_mesh():
    return plsc.VectorSubcoreMesh(core_axis_name="c", subcore_axis_name="s")


def _deg_call(col, zeros_n1, ones_ch):
    dw = ones_ch.shape[1]
    @functools.partial(
        pl.kernel,
        out_type=jax.ShapeDtypeStruct((NC, N2, dw), jnp.float32),
        mesh=_sc_mesh(),
        scratch_types=(
            [pltpu.VMEM((CH,), jnp.int32) for _ in range(NBUF)]
            + [pltpu.VMEM((CH, dw), jnp.float32),
               pltpu.VMEM_SHARED((N2, dw), jnp.float32)]
            + [pltpu.SemaphoreType.DMA for _ in range(NBUF)]
        ),
    )
    def deg_kernel(col_hbm, zeros_hbm, ones_hbm, out_hbm, *refs):
        cidx = refs[0:NBUF]
        ones_v = refs[NBUF]
        deg_sh = refs[NBUF + 1]
        ssem = refs[NBUF + 2:NBUF + 2 + NBUF]
        c = lax.axis_index("c")
        s = lax.axis_index("s")
        wid = s * NC + c
        base = wid * EPW
        pltpu.sync_copy(ones_hbm, ones_v)
        pltpu.sync_copy(zeros_hbm.at[pl.ds(RPT2 * s, RPT2)],
                        deg_sh.at[pl.ds(RPT2 * s, RPT2)])
        plsc.subcore_barrier()

        def fetch(g, b):
            eb = pl.multiple_of(base + g * CH, 8)
            pltpu.sync_copy(col_hbm.at[pl.ds(eb, CH)], cidx[b])

        def sstart(b):
            pltpu.async_copy(ones_v, deg_sh.at[cidx[b]], ssem[b], add=True)

        def swait(b):
            pltpu.make_async_copy(ones_v, deg_sh.at[cidx[b]], ssem[b]).wait()

        for b in range(NBUF - 1):
            fetch(b, b)

        def body(k, carry):
            g0 = NBUF * k
            for j in range(NBUF):
                g = g0 + j
                sstart(j)
                tgt = (j + NBUF - 1) % NBUF
                if j == 0:
                    @pl.when(k == 0)
                    def _():
                        fetch(NBUF - 1, tgt)

                    @pl.when(k >= 1)
                    def _():
                        swait(tgt)
                        fetch(g + NBUF - 1, tgt)
                else:
                    @pl.when(g + NBUF - 1 < NCHUNK)
                    def _():
                        swait(tgt)
                        fetch(g + NBUF - 1, tgt)
            return carry

        lax.fori_loop(0, NCHUNK // NBUF, body, 0)
        for t in range(NCHUNK % NBUF):
            g = (NCHUNK // NBUF) * NBUF + t
            sstart(g % NBUF)
        for g in range(NCHUNK - NBUF, NCHUNK):
            swait(g % NBUF)
        plsc.subcore_barrier()
        pltpu.sync_copy(deg_sh.at[pl.ds(RPT2 * s, RPT2)],
                        out_hbm.at[c, pl.ds(RPT2 * s, RPT2)])

    return deg_kernel(col, zeros_n1, ones_ch)


def _edge_call(hp, row, col, zeros_nd):
    @functools.partial(
        pl.kernel,
        out_type=jax.ShapeDtypeStruct((NC, N2, D), jnp.float32),
        mesh=_sc_mesh(),
        scratch_types=(
            [pltpu.VMEM((CH,), jnp.int32) for _ in range(NBUF)]
            + [pltpu.VMEM((CH,), jnp.int32) for _ in range(NBUF)]
            + [pltpu.VMEM((CH, D), jnp.float32) for _ in range(NBUF)]
            + [pltpu.VMEM_SHARED((N2, D), jnp.float32)]
            + [pltpu.SemaphoreType.DMA for _ in range(2 * NBUF)]
        ),
    )
    def edge_kernel(hp_hbm, row_hbm, col_hbm, zeros_hbm, out_hbm, *refs):
        ridx = refs[0:NBUF]
        cidx = refs[NBUF:2 * NBUF]
        rows = refs[2 * NBUF:3 * NBUF]
        acc_sh = refs[3 * NBUF]
        gsem = refs[3 * NBUF + 1:3 * NBUF + 1 + NBUF]
        ssem = refs[3 * NBUF + 1 + NBUF:3 * NBUF + 1 + 2 * NBUF]
        c = lax.axis_index("c")
        s = lax.axis_index("s")
        wid = s * NC + c
        base = wid * EPW
        pltpu.sync_copy(zeros_hbm.at[pl.ds(RPT * s, RPT)],
                        acc_sh.at[pl.ds(RPT * s, RPT)])
        plsc.subcore_barrier()

        def fetch(g, b):
            eb = pl.multiple_of(base + g * CH, 8)
            pltpu.sync_copy(row_hbm.at[pl.ds(eb, CH)], ridx[b])
            pltpu.sync_copy(col_hbm.at[pl.ds(eb, CH)], cidx[b])
            pltpu.async_copy(hp_hbm.at[ridx[b]], rows[b], gsem[b])

        def gwait(b):
            pltpu.make_async_copy(hp_hbm.at[ridx[b]], rows[b], gsem[b]).wait()

        def sstart(b):
            pltpu.async_copy(rows[b], acc_sh.at[cidx[b]], ssem[b], add=True)

        def swait(b):
            pltpu.make_async_copy(rows[b], acc_sh.at[cidx[b]], ssem[b]).wait()

        for b in range(NBUF - 1):
            fetch(b, b)

        def body(k, carry):
            g0 = NBUF * k
            for j in range(NBUF):
                g = g0 + j
                gwait(j)
                sstart(j)
                tgt = (j + NBUF - 1) % NBUF
                if j == 0:
                    @pl.when(k == 0)
                    def _():
                        fetch(NBUF - 1, tgt)

                    @pl.when(k >= 1)
                    def _():
                        swait(tgt)
                        fetch(g + NBUF - 1, tgt)
                else:
                    @pl.when(g + NBUF - 1 < NCHUNK)
                    def _():
                        swait(tgt)
                        fetch(g + NBUF - 1, tgt)
            return carry

        lax.fori_loop(0, NCHUNK // NBUF, body, 0)
        for t in range(NCHUNK % NBUF):
            g = (NCHUNK // NBUF) * NBUF + t
            gwait(g % NBUF)
            sstart(g % NBUF)
        for g in range(NCHUNK - NBUF, NCHUNK):
            swait(g % NBUF)
        plsc.subcore_barrier()
        pltpu.sync_copy(acc_sh.at[pl.ds(RPT * s, RPT)],
                        out_hbm.at[c, pl.ds(RPT * s, RPT)])

    return edge_kernel(hp, row, col, zeros_nd)


def _dis_from(degp_ref):
    deg = (degp_ref[0] + degp_ref[1])[:N, 0:1]
    return jnp.where(deg > 0.0, lax.rsqrt(deg), 0.0)


def _bn(y, g_ref, b_ref):
    mean = jnp.mean(y, axis=0, keepdims=True)
    var = jnp.mean((y - mean) ** 2, axis=0, keepdims=True)
    return (y - mean) * lax.rsqrt(var + EPS) * g_ref[...][None, :] + b_ref[...][None, :]


def _mm2_body(h_ref, wg_ref, wl_ref, g_ref, l_ref):
    h = h_ref[...]
    g_ref[...] = jnp.dot(h, wg_ref[...], preferred_element_type=jnp.float32)
    l_ref[...] = jnp.dot(h, wl_ref[...], preferred_element_type=jnp.float32)


def _mm2_call(h, wg, wl):
    return pl.pallas_call(
        _mm2_body,
        out_shape=(jax.ShapeDtypeStruct((N, D), jnp.float32),
                   jax.ShapeDtypeStruct((N, D), jnp.float32)),
    )(h, wg, wl)


def _scale_body(g0_ref, degp_ref, hp_ref, dis_ref):
    dis = _dis_from(degp_ref)
    hp_ref[...] = g0_ref[...] * dis
    dis_ref[...] = dis


def _scale_call(g0, degp):
    return pl.pallas_call(
        _scale_body,
        out_shape=(jax.ShapeDtypeStruct((N, D), jnp.float32),
                   jax.ShapeDtypeStruct((N, 1), jnp.float32)),
    )(g0, degp)


def _mida_body(l0_ref, acc_ref, dis_ref, g_ref, b_ref, wg_ref,
               h1_ref, hp1_ref):
    dis = dis_ref[...]
    y = l0_ref[...] + dis * (acc_ref[0] + acc_ref[1])[:N]
    h = jnp.maximum(_bn(y, g_ref, b_ref), 0.0)
    h1_ref[...] = h
    hp1_ref[...] = jnp.dot(h, wg_ref[...], preferred_element_type=jnp.float32) * dis


def _mida_call(l0, acc, dis, gamma, beta, wg):
    return pl.pallas_call(
        _mida_body,
        out_shape=(jax.ShapeDtypeStruct((N, D), jnp.float32),
                   jax.ShapeDtypeStruct((N, D), jnp.float32)),
    )(l0, acc, dis, gamma, beta, wg)


def _midb_body(h1_ref, wl_ref, l1_ref):
    l1_ref[...] = jnp.dot(h1_ref[...], wl_ref[...],
                          preferred_element_type=jnp.float32)


def _midb_call(h1, wl):
    return pl.pallas_call(
        _midb_body,
        out_shape=jax.ShapeDtypeStruct((N, D), jnp.float32),
    )(h1, wl)


def _fin_body(l1_ref, acc_ref, dis_ref, g_ref, b_ref, out_ref):
    dis = dis_ref[...]
    y = l1_ref[...] + dis * (acc_ref[0] + acc_ref[1])[:N]
    out_ref[...] = _bn(y, g_ref, b_ref)


def _fin_call(l1, acc, dis, gamma, beta):
    return pl.pallas_call(
        _fin_body,
        out_shape=jax.ShapeDtypeStruct((N, D), jnp.float32),
    )(l1, acc, dis, gamma, beta)


def kernel(x, edge_index, W_lin0, W_gcn0, gamma0, beta0,
           W_lin1, W_gcn1, gamma1, beta1):
    row = edge_index[0]
    col = edge_index[1]
    zeros_nd = jnp.zeros((N2, D), jnp.float32)
    zeros_n1 = jnp.zeros((N2, 128), jnp.float32)
    ones_ch = jnp.ones((CH, 128), jnp.float32)

    degp = _deg_call(col, zeros_n1, ones_ch)
    g0, l0 = _mm2_call(x, W_gcn0, W_lin0)
    hp0, dis = _scale_call(g0, degp)
    acc0 = _edge_call(hp0, row, col, zeros_nd)
    h1, hp1 = _mida_call(l0, acc0, dis, gamma0, beta0, W_gcn1)
    acc1 = _edge_call(hp1, row, col, zeros_nd)
    l1 = _midb_call(h1, W_lin1)
    return _fin_call(l1, acc1, dis, gamma1, beta1)
